# baseline (speedup 1.0000x reference)
import functools

import jax
import jax.numpy as jnp
from jax import lax
from jax.experimental import pallas as pl
from jax.experimental.pallas import tpu as pltpu
from jax.experimental.pallas import tpu_sc as plsc

_G = 128
_C = 256
_NSL = _C // 16

_SC_ROWS = 40960
_SC_OFF = 50000 - _SC_ROWS
_NW = 32
_RPW = _SC_ROWS // _NW
_CH = 256
_NCH = _RPW // _CH

_NSTEPS = 10
_RB = _SC_OFF // _NSTEPS
_RG = 50000 // _NSTEPS


def _sc_body(b_hbm, ids_hbm, w1_hbm, sums_hbm, cnts_hbm,
             buf, idsv, w1v, sums_v, cnts_v):
    wid = lax.axis_index("s") * 2 + lax.axis_index("c")
    base = _SC_OFF + wid * _RPW

    pltpu.sync_copy(w1_hbm, w1v)
    zf = jnp.zeros((16,), jnp.float32)
    zi = jnp.zeros((16,), jnp.int32)
    for gi in range(_G):
        sums_v[gi, pl.ds(0, 16)] = zf
        cnts_v[gi, pl.ds(0, 16)] = zi
    w1r = [w1v[pl.ds(j * 16, 16)] for j in range(_NSL)]

    def flush(g, cnt, acc):
        @pl.when(g >= 0)
        def _():
            v = acc[0] * w1r[0]
            for j in range(1, _NSL):
                v = v + acc[j] * w1r[j]
            sums_v[g, pl.ds(0, 16)] = v
            cnts_v[g, pl.ds(0, 16)] = jnp.full((16,), cnt, jnp.int32)

    def row_body(r, carry):
        g = carry[0]
        cnt = carry[1]
        acc = carry[2:]
        gr = idsv[pl.ds(r, 16)][0]
        changed = gr != g
        @pl.when(changed)
        def _():
            flush(g, cnt, acc)
        acc = [jnp.where(changed, zf, a) for a in acc]
        cnt = jnp.where(changed, 0, cnt)
        acc = [a + buf[r, pl.ds(j * 16, 16)] for j, a in enumerate(acc)]
        return (gr, cnt + 1) + tuple(acc)

    carry = (jnp.int32(-1), jnp.int32(0)) + tuple(zf for _ in range(_NSL))
    for ch in range(_NCH):
        start = base + ch * _CH
        pltpu.sync_copy(b_hbm.at[pl.ds(start, _CH), :], buf)
        pltpu.sync_copy(ids_hbm.at[pl.ds(start, _CH)], idsv.at[pl.ds(0, _CH)])
        carry = lax.fori_loop(0, _CH, row_body, carry)
    flush(carry[0], carry[1], carry[2:])

    pltpu.sync_copy(sums_v, sums_hbm.at[wid])
    pltpu.sync_copy(cnts_v, cnts_hbm.at[wid])


def _sc_partials(B_z, ids_b, w1):
    fn = functools.partial(
        pl.kernel,
        mesh=plsc.VectorSubcoreMesh(core_axis_name="c", subcore_axis_name="s"),
        out_type=[jax.ShapeDtypeStruct((_NW, _G, 16), jnp.float32),
                  jax.ShapeDtypeStruct((_NW, _G, 16), jnp.int32)],
        scratch_types=[pltpu.VMEM((_CH, _C), jnp.float32),
                       pltpu.VMEM((_CH + 16,), jnp.int32),
                       pltpu.VMEM((_C,), jnp.float32),
                       pltpu.VMEM((_G, 16), jnp.float32),
                       pltpu.VMEM((_G, 16), jnp.int32)],
    )(_sc_body)
    return fn(B_z, ids_b, w1)


def _tc_main_body(ib_ref, ig_ref, bsh_ref, g_ref, w_ref, accb_ref, accg_ref):
    i = pl.program_id(0)

    @pl.when(i == 0)
    def _init():
        accb_ref[...] = jnp.zeros_like(accb_ref)
        accg_ref[...] = jnp.zeros_like(accg_ref)

    w1 = w_ref[0, :_C]
    w2 = w_ref[0, _C:]
    sv_b = jnp.sum(bsh_ref[...] * w1[None, :], axis=1, keepdims=True)
    sv_g = jnp.sum(g_ref[...] * w2[None, :], axis=1, keepdims=True)
    svc_b = jnp.concatenate([sv_b, jnp.ones_like(sv_b)], axis=1)
    svc_g = jnp.concatenate([sv_g, jnp.ones_like(sv_g)], axis=1)
    ids_b = ib_ref[0]
    ids_g = ig_ref[0]
    seg_b = lax.broadcasted_iota(jnp.int32, (_G, _RB), 0)
    seg_g = lax.broadcasted_iota(jnp.int32, (_G, _RG), 0)
    oh_b = (seg_b == ids_b).astype(jnp.float32)
    oh_g = (seg_g == ids_g).astype(jnp.float32)
    dn = (((1,), (0,)), ((), ()))
    accb_ref[...] += lax.dot_general(
        oh_b, svc_b, dn, preferred_element_type=jnp.float32)
    accg_ref[...] += lax.dot_general(
        oh_g, svc_g, dn, preferred_element_type=jnp.float32)


def _tc_main(ids_b_sh, ids_g, B_z, G_z, W):
    return pl.pallas_call(
        _tc_main_body,
        grid=(_NSTEPS,),
        in_specs=[
            pl.BlockSpec((1, 1, _RB), lambda i: (i, 0, 0)),
            pl.BlockSpec((1, 1, _RG), lambda i: (i, 0, 0)),
            pl.BlockSpec((_RB, _C), lambda i: (i, 0)),
            pl.BlockSpec((_RG, _C), lambda i: (i, 0)),
            pl.BlockSpec((1, 2 * _C), lambda i: (0, 0)),
        ],
        out_specs=[pl.BlockSpec((_G, 2), lambda i: (0, 0)),
                   pl.BlockSpec((_G, 2), lambda i: (0, 0))],
        out_shape=[jax.ShapeDtypeStruct((_G, 2), jnp.float32),
                   jax.ShapeDtypeStruct((_G, 2), jnp.float32)],
        compiler_params=pltpu.CompilerParams(
            dimension_semantics=("arbitrary",)),
    )(ids_b_sh, ids_g, B_z, G_z, W)


def _epi_body(scs_ref, scc_ref, accb_ref, accg_ref, bias_ref, out_ref):
    scs = jnp.sum(scs_ref[...], axis=(0, 2))
    scc = jnp.sum(scc_ref[...], axis=(0, 2)).astype(jnp.float32) / 16.0
    bsum = accb_ref[:, 0] + scs
    bcnt = accb_ref[:, 1] + scc
    res = (bsum / jnp.maximum(bcnt, 1.0)
           + accg_ref[:, 0] / jnp.maximum(accg_ref[:, 1], 1.0)
           + bias_ref[0, 0])
    out_ref[...] = res[:, None]


def _epilogue(sc_sums, sc_cnts, accb, accg, bias):
    return pl.pallas_call(
        _epi_body,
        out_shape=jax.ShapeDtypeStruct((_G, 1), jnp.float32),
    )(sc_sums, sc_cnts, accb, accg, bias)


def kernel(B_z, G_z, x_b_batch, x_g_batch, W, b):
    ids_b = x_b_batch.astype(jnp.int32)
    ids_g = x_g_batch.astype(jnp.int32)
    w1 = W[0, :_C]
    sc_sums, sc_cnts = _sc_partials(B_z, ids_b, w1)
    ids_b_sh = ids_b[:_SC_OFF].reshape(_NSTEPS, 1, _RB)
    ids_g_r = ids_g.reshape(_NSTEPS, 1, _RG)
    accb, accg = _tc_main(ids_b_sh, ids_g_r, B_z, G_z, W)
    return _epilogue(sc_sums, sc_cnts, accb, accg, b.reshape(1, 1))


# speedup vs baseline: 2.3862x; 2.3862x over previous
import functools

import jax
import jax.numpy as jnp
from jax.experimental import pallas as pl
from jax.experimental.pallas import tpu as pltpu

_G = 128
_C = 256


def _pool_kernel(ib_ref, ig_ref, b_ref, g_ref, w_ref, bias_ref, out_ref,
                 accb_ref, accg_ref, *, nsteps):
    i = pl.program_id(0)

    @pl.when(i == 0)
    def _init():
        accb_ref[...] = jnp.zeros_like(accb_ref)
        accg_ref[...] = jnp.zeros_like(accg_ref)

    w1 = w_ref[0, :_C]
    w2 = w_ref[0, _C:]
    bb = b_ref[...]
    gb = g_ref[...]
    sv_b = jnp.sum(bb * w1[None, :], axis=1, keepdims=True)
    sv_g = jnp.sum(gb * w2[None, :], axis=1, keepdims=True)
    ones = jnp.ones_like(sv_b)
    svc_b = jnp.concatenate([sv_b, ones], axis=1)
    svc_g = jnp.concatenate([sv_g, ones], axis=1)
    ids_b = ib_ref[0]
    ids_g = ig_ref[0]
    seg = jax.lax.broadcasted_iota(jnp.int32, (_G, ids_b.shape[1]), 0)
    oh_b = (seg == ids_b).astype(jnp.float32)
    oh_g = (seg == ids_g).astype(jnp.float32)
    dn = (((1,), (0,)), ((), ()))
    accb_ref[...] += jax.lax.dot_general(
        oh_b, svc_b, dn, preferred_element_type=jnp.float32)
    accg_ref[...] += jax.lax.dot_general(
        oh_g, svc_g, dn, preferred_element_type=jnp.float32)

    @pl.when(i == nsteps - 1)
    def _fin():
        ab = accb_ref[...]
        ag = accg_ref[...]
        res = (ab[:, 0] / jnp.maximum(ab[:, 1], 1.0)
               + ag[:, 0] / jnp.maximum(ag[:, 1], 1.0)
               + bias_ref[0, 0])
        out_ref[...] = res[:, None]


def kernel(B_z, G_z, x_b_batch, x_g_batch, W, b):
    nb, c = B_z.shape
    r = 5000
    nsteps = nb // r
    ib = x_b_batch.astype(jnp.int32).reshape(nsteps, 1, r)
    ig = x_g_batch.astype(jnp.int32).reshape(nsteps, 1, r)
    bias = b.reshape(1, 1)
    out = pl.pallas_call(
        functools.partial(_pool_kernel, nsteps=nsteps),
        grid=(nsteps,),
        in_specs=[
            pl.BlockSpec((1, 1, r), lambda i: (i, 0, 0)),
            pl.BlockSpec((1, 1, r), lambda i: (i, 0, 0)),
            pl.BlockSpec((r, c), lambda i: (i, 0)),
            pl.BlockSpec((r, c), lambda i: (i, 0)),
            pl.BlockSpec((1, 2 * _C), lambda i: (0, 0)),
            pl.BlockSpec((1, 1), lambda i: (0, 0)),
        ],
        out_specs=pl.BlockSpec((_G, 1), lambda i: (0, 0)),
        out_shape=jax.ShapeDtypeStruct((_G, 1), jnp.float32),
        scratch_shapes=[pltpu.VMEM((_G, 2), jnp.float32),
                        pltpu.VMEM((_G, 2), jnp.float32)],
        compiler_params=pltpu.CompilerParams(
            dimension_semantics=("arbitrary",)),
    )(ib, ig, B_z, G_z, W, bias)
    return out
